# baseline (speedup 1.0000x reference)
import functools
import math

import jax
import jax.numpy as jnp
from jax import lax
from jax.experimental import pallas as pl
from jax.experimental.pallas import tpu as pltpu
from jax.experimental.pallas import tpu_sc as plsc

_SIZE = 32000
_N = 2048
_EPS = 0.1 / (_SIZE - 2)
_CONF = 0.9
_C_ROW = _CONF * math.log(_CONF) + 0.1 * math.log(_EPS)

_INFO = plsc.get_sparse_core_info()
_NC, _NS, _L = _INFO.num_cores, _INFO.num_subcores, _INFO.num_lanes
_NW = _NC * _NS
_BPW = _N // _NW

_SC_ROWS = 512
_TC_ROWS = _N - _SC_ROWS
_RPT = _SC_ROWS // _NW
_BCW = 3200
_NCH = _SIZE // _BCW

_BR = 256
_BC = 6400
_R = _TC_ROWS // _BR
_C = _SIZE // _BC


def _dense_body(x_ref, tgt_ref, o_ref):
    r = pl.program_id(0)
    c = pl.program_id(1)
    first = jnp.logical_and(r == 0, c == 0)

    xb = x_ref[...]
    tgt = tgt_ref[0, 0, :]
    maskf = (tgt != 0).astype(jnp.float32)

    msum = jnp.sum(jnp.sum(xb, axis=1) * maskf)
    m0 = jnp.sum(xb[:, 0] * maskf)
    c0_term = jnp.where(c == 0, _EPS * m0, 0.0)

    contrib = (-_EPS * msum + c0_term).reshape(1, 1)
    o_ref[...] = jnp.where(first, contrib, o_ref[...] + contrib)


def _sc_body(x2d, tgt, out_a, out_m, out_d,
             tgt_v, win_v, dt_v, dbuf, c0buf, acc_v, cnt_v, tot_v, sem):
    wid = lax.axis_index("s") * _NC + lax.axis_index("c")
    lanes = lax.broadcasted_iota(jnp.int32, (_L,), 0)

    base = wid * _BPW
    pltpu.sync_copy(tgt.at[pl.ds(base, _BPW)], tgt_v)
    copies = []
    for k in range(_BPW // _L):
        t16 = tgt_v[pl.ds(k * _L, _L)]
        for l in range(_L):
            r = k * _L + l
            t_r = t16[l]
            cb = (t_r // _L) * _L
            copies.append(pltpu.make_async_copy(
                x2d.at[base + r, pl.ds(cb, _L)],
                win_v.at[pl.ds(r * _L, _L)],
                sem,
            ))
    for cp in copies:
        cp.start()
    for cp in copies:
        cp.wait()
    acc = jnp.zeros((_L,), jnp.float32)
    cnt = jnp.zeros((_L,), jnp.float32)
    for k in range(_BPW // _L):
        t16 = tgt_v[pl.ds(k * _L, _L)]
        m = t16 != 0
        cnt = cnt + jnp.where(m, 1.0, 0.0)
        for l in range(_L):
            r = k * _L + l
            t_r = t16[l]
            v = win_v[pl.ds(r * _L, _L)]
            sel_f = jnp.where(lanes == t_r % _L, 1.0, 0.0)
            mask_f = jnp.where(t_r != 0, 1.0, 0.0)
            acc = acc + v * sel_f * mask_f
    acc_v[...] = acc
    cnt_v[...] = cnt
    pltpu.sync_copy(acc_v, out_a.at[wid])
    pltpu.sync_copy(cnt_v, out_m.at[wid])

    dbase = _TC_ROWS + wid * _RPT
    pltpu.sync_copy(tgt.at[pl.ds(dbase, _L)], dt_v)
    dt = dt_v[...]
    mfs = [jnp.where(dt[r] != 0, 1.0, 0.0) for r in range(_RPT)]

    def chunk_body(c, total):
        pltpu.sync_copy(x2d.at[pl.ds(dbase, _RPT), pl.ds(c * _BCW, _BCW)],
                        dbuf)

        def j_body(j, tot):
            for r in range(_RPT):
                tot = tot + dbuf[r, pl.ds(j * _L, _L)] * mfs[r]
            return tot

        return lax.fori_loop(0, _BCW // _L, j_body, total)

    total = lax.fori_loop(0, _NCH, chunk_body, jnp.zeros((_L,), jnp.float32))

    pltpu.sync_copy(x2d.at[pl.ds(dbase, _RPT), pl.ds(0, 128)], c0buf)
    lane0 = jnp.where(lanes == 0, 1.0, 0.0)
    for r in range(_RPT):
        total = total - c0buf[r, pl.ds(0, _L)] * lane0 * mfs[r]
    tot_v[...] = total
    pltpu.sync_copy(tot_v, out_d.at[wid])


_sc_kernel = pl.kernel(
    _sc_body,
    out_type=[
        jax.ShapeDtypeStruct((_NW, _L), jnp.float32),
        jax.ShapeDtypeStruct((_NW, _L), jnp.float32),
        jax.ShapeDtypeStruct((_NW, _L), jnp.float32),
    ],
    mesh=plsc.VectorSubcoreMesh(core_axis_name="c", subcore_axis_name="s"),
    scratch_types=[
        pltpu.VMEM((_BPW,), jnp.int32),
        pltpu.VMEM((_BPW * _L,), jnp.float32),
        pltpu.VMEM((_L,), jnp.int32),
        pltpu.VMEM((_RPT, _BCW), jnp.float32),
        pltpu.VMEM((_RPT, 128), jnp.float32),
        pltpu.VMEM((_L,), jnp.float32),
        pltpu.VMEM((_L,), jnp.float32),
        pltpu.VMEM((_L,), jnp.float32),
        pltpu.SemaphoreType.DMA,
    ],
)


def kernel(x, target):
    tgt = target.astype(jnp.int32)
    out_a, out_m, out_d = _sc_kernel(x, tgt)
    dense = pl.pallas_call(
        _dense_body,
        grid=(_R, _C),
        in_specs=[
            pl.BlockSpec((_BR, _BC), lambda r, c: (r, c)),
            pl.BlockSpec((1, 1, _BR), lambda r, c: (r, 0, 0)),
        ],
        out_specs=pl.BlockSpec((1, 1), lambda r, c: (0, 0)),
        out_shape=jax.ShapeDtypeStruct((1, 1), jnp.float32),
    )(x, tgt[:_TC_ROWS].reshape(_R, 1, _BR))
    a = jnp.sum(out_a)
    m = jnp.sum(out_m)
    d_sc = jnp.sum(out_d)
    return (_C_ROW * m - (_CONF - _EPS) * a - _EPS * d_sc
            + dense[0, 0]).reshape(())


# speedup vs baseline: 1.0480x; 1.0480x over previous
import functools
import math

import jax
import jax.numpy as jnp
from jax import lax
from jax.experimental import pallas as pl
from jax.experimental.pallas import tpu as pltpu
from jax.experimental.pallas import tpu_sc as plsc

_SIZE = 32000
_N = 2048
_EPS = 0.1 / (_SIZE - 2)
_CONF = 0.9
_C_ROW = _CONF * math.log(_CONF) + 0.1 * math.log(_EPS)

_INFO = plsc.get_sparse_core_info()
_NC, _NS, _L = _INFO.num_cores, _INFO.num_subcores, _INFO.num_lanes
_NW = _NC * _NS
_BPW = _N // _NW

_SC_ROWS = 512
_TC_ROWS = _N - _SC_ROWS
_RPT = _SC_ROWS // _NW
_BCW = 3200
_NCH = _SIZE // _BCW

_BR = 256
_BC = 6400
_R = _TC_ROWS // _BR
_C = _SIZE // _BC


def _dense_body(x_ref, tgt_ref, o_ref):
    r = pl.program_id(0)
    c = pl.program_id(1)
    first = jnp.logical_and(r == 0, c == 0)

    xb = x_ref[...]
    tgt = tgt_ref[0, 0, :]
    maskf = (tgt != 0).astype(jnp.float32)

    msum = jnp.sum(jnp.sum(xb, axis=1) * maskf)
    m0 = jnp.sum(xb[:, 0] * maskf)
    c0_term = jnp.where(c == 0, _EPS * m0, 0.0)

    contrib = (-_EPS * msum + c0_term).reshape(1, 1)
    o_ref[...] = jnp.where(first, contrib, o_ref[...] + contrib)


def _sc_body(x2d, tgt, out_a, out_m, out_d,
             tgt_v, win_v, dt_v, dbuf, dbuf2, c0buf, acc_v, cnt_v, tot_v,
             sem, semd, semd2):
    wid = lax.axis_index("s") * _NC + lax.axis_index("c")
    lanes = lax.broadcasted_iota(jnp.int32, (_L,), 0)

    base = wid * _BPW
    pltpu.sync_copy(tgt.at[pl.ds(base, _BPW)], tgt_v)
    copies = []
    for k in range(_BPW // _L):
        t16 = tgt_v[pl.ds(k * _L, _L)]
        for l in range(_L):
            r = k * _L + l
            t_r = t16[l]
            cb = (t_r // _L) * _L
            copies.append(pltpu.make_async_copy(
                x2d.at[base + r, pl.ds(cb, _L)],
                win_v.at[pl.ds(r * _L, _L)],
                sem,
            ))
    for cp in copies:
        cp.start()
    for cp in copies:
        cp.wait()
    acc = jnp.zeros((_L,), jnp.float32)
    cnt = jnp.zeros((_L,), jnp.float32)
    for k in range(_BPW // _L):
        t16 = tgt_v[pl.ds(k * _L, _L)]
        m = t16 != 0
        cnt = cnt + jnp.where(m, 1.0, 0.0)
        for l in range(_L):
            r = k * _L + l
            t_r = t16[l]
            v = win_v[pl.ds(r * _L, _L)]
            sel_f = jnp.where(lanes == t_r % _L, 1.0, 0.0)
            mask_f = jnp.where(t_r != 0, 1.0, 0.0)
            acc = acc + v * sel_f * mask_f
    acc_v[...] = acc
    cnt_v[...] = cnt
    pltpu.sync_copy(acc_v, out_a.at[wid])
    pltpu.sync_copy(cnt_v, out_m.at[wid])

    dbase = _TC_ROWS + wid * _RPT
    pltpu.sync_copy(tgt.at[pl.ds(dbase, _L)], dt_v)
    dt = dt_v[...]
    mfs = [jnp.where(dt[r] != 0, 1.0, 0.0) for r in range(_RPT)]

    bufs = (dbuf, dbuf2)
    sems = (semd, semd2)

    def chunk_cp(c):
        return pltpu.make_async_copy(
            x2d.at[pl.ds(dbase, _RPT), pl.ds(c * _BCW, _BCW)],
            bufs[c % 2], sems[c % 2])

    accs = tuple(jnp.zeros((_L,), jnp.float32) for _ in range(8))
    pending = chunk_cp(0)
    pending.start()
    for c in range(_NCH):
        if c + 1 < _NCH:
            nxt = chunk_cp(c + 1)
            nxt.start()
        pending.wait()
        cur = bufs[c % 2]

        def j_body(j, a, cur=cur):
            a = list(a)
            for r in range(_RPT):
                a[r % 8] = a[r % 8] + cur[r, pl.ds(j * _L, _L)] * mfs[r]
            return tuple(a)

        accs = lax.fori_loop(0, _BCW // _L, j_body, accs)
        if c + 1 < _NCH:
            pending = nxt
    total = accs[0]
    for q in range(1, 8):
        total = total + accs[q]

    pltpu.sync_copy(x2d.at[pl.ds(dbase, _RPT), pl.ds(0, 128)], c0buf)
    lane0 = jnp.where(lanes == 0, 1.0, 0.0)
    for r in range(_RPT):
        total = total - c0buf[r, pl.ds(0, _L)] * lane0 * mfs[r]
    tot_v[...] = total
    pltpu.sync_copy(tot_v, out_d.at[wid])


_sc_kernel = pl.kernel(
    _sc_body,
    out_type=[
        jax.ShapeDtypeStruct((_NW, _L), jnp.float32),
        jax.ShapeDtypeStruct((_NW, _L), jnp.float32),
        jax.ShapeDtypeStruct((_NW, _L), jnp.float32),
    ],
    mesh=plsc.VectorSubcoreMesh(core_axis_name="c", subcore_axis_name="s"),
    scratch_types=[
        pltpu.VMEM((_BPW,), jnp.int32),
        pltpu.VMEM((_BPW * _L,), jnp.float32),
        pltpu.VMEM((_L,), jnp.int32),
        pltpu.VMEM((_RPT, _BCW), jnp.float32),
        pltpu.VMEM((_RPT, _BCW), jnp.float32),
        pltpu.VMEM((_RPT, 128), jnp.float32),
        pltpu.VMEM((_L,), jnp.float32),
        pltpu.VMEM((_L,), jnp.float32),
        pltpu.VMEM((_L,), jnp.float32),
        pltpu.SemaphoreType.DMA,
        pltpu.SemaphoreType.DMA,
        pltpu.SemaphoreType.DMA,
    ],
)


def kernel(x, target):
    tgt = target.astype(jnp.int32)
    out_a, out_m, out_d = _sc_kernel(x, tgt)
    dense = pl.pallas_call(
        _dense_body,
        grid=(_R, _C),
        in_specs=[
            pl.BlockSpec((_BR, _BC), lambda r, c: (r, c)),
            pl.BlockSpec((1, 1, _BR), lambda r, c: (r, 0, 0)),
        ],
        out_specs=pl.BlockSpec((1, 1), lambda r, c: (0, 0)),
        out_shape=jax.ShapeDtypeStruct((1, 1), jnp.float32),
    )(x, tgt[:_TC_ROWS].reshape(_R, 1, _BR))
    a = jnp.sum(out_a)
    m = jnp.sum(out_m)
    d_sc = jnp.sum(out_d)
    return (_C_ROW * m - (_CONF - _EPS) * a - _EPS * d_sc
            + dense[0, 0]).reshape(())


# speedup vs baseline: 1.0676x; 1.0187x over previous
import functools
import math

import jax
import jax.numpy as jnp
from jax import lax
from jax.experimental import pallas as pl
from jax.experimental.pallas import tpu as pltpu
from jax.experimental.pallas import tpu_sc as plsc

_SIZE = 32000
_N = 2048
_EPS = 0.1 / (_SIZE - 2)
_CONF = 0.9
_C_ROW = _CONF * math.log(_CONF) + 0.1 * math.log(_EPS)

_INFO = plsc.get_sparse_core_info()
_NC, _NS, _L = _INFO.num_cores, _INFO.num_subcores, _INFO.num_lanes
_NW = _NC * _NS
_BPW = _N // _NW

_SC_ROWS = 256
_TC_ROWS = _N - _SC_ROWS
_RPT = _SC_ROWS // _NW
_BCW = 3200
_NCH = _SIZE // _BCW

_BR = 256
_BC = 6400
_R = _TC_ROWS // _BR
_C = _SIZE // _BC


def _dense_body(x_ref, tgt_ref, o_ref):
    r = pl.program_id(0)
    c = pl.program_id(1)
    first = jnp.logical_and(r == 0, c == 0)

    xb = x_ref[...]
    tgt = tgt_ref[0, 0, :]
    maskf = (tgt != 0).astype(jnp.float32)

    msum = jnp.sum(jnp.sum(xb, axis=1) * maskf)
    m0 = jnp.sum(xb[:, 0] * maskf)
    c0_term = jnp.where(c == 0, _EPS * m0, 0.0)

    contrib = (-_EPS * msum + c0_term).reshape(1, 1)
    o_ref[...] = jnp.where(first, contrib, o_ref[...] + contrib)


def _sc_body(x2d, tgt, out_a, out_m, out_d,
             tgt_v, win_v, dt_v, dbuf, dbuf2, c0buf, acc_v, cnt_v, tot_v,
             sem, semd, semd2):
    wid = lax.axis_index("s") * _NC + lax.axis_index("c")
    lanes = lax.broadcasted_iota(jnp.int32, (_L,), 0)

    base = wid * _BPW
    pltpu.sync_copy(tgt.at[pl.ds(base, _BPW)], tgt_v)
    copies = []
    for k in range(_BPW // _L):
        t16 = tgt_v[pl.ds(k * _L, _L)]
        for l in range(_L):
            r = k * _L + l
            t_r = t16[l]
            cb = (t_r // _L) * _L
            copies.append(pltpu.make_async_copy(
                x2d.at[base + r, pl.ds(cb, _L)],
                win_v.at[pl.ds(r * _L, _L)],
                sem,
            ))
    for cp in copies:
        cp.start()
    for cp in copies:
        cp.wait()
    acc = jnp.zeros((_L,), jnp.float32)
    cnt = jnp.zeros((_L,), jnp.float32)
    for k in range(_BPW // _L):
        t16 = tgt_v[pl.ds(k * _L, _L)]
        m = t16 != 0
        cnt = cnt + jnp.where(m, 1.0, 0.0)
        for l in range(_L):
            r = k * _L + l
            t_r = t16[l]
            v = win_v[pl.ds(r * _L, _L)]
            sel_f = jnp.where(lanes == t_r % _L, 1.0, 0.0)
            mask_f = jnp.where(t_r != 0, 1.0, 0.0)
            acc = acc + v * sel_f * mask_f
    acc_v[...] = acc
    cnt_v[...] = cnt
    pltpu.sync_copy(acc_v, out_a.at[wid])
    pltpu.sync_copy(cnt_v, out_m.at[wid])

    dbase = _TC_ROWS + wid * _RPT
    pltpu.sync_copy(tgt.at[pl.ds(dbase, _L)], dt_v)
    dt = dt_v[...]
    mfs = [jnp.where(dt[r] != 0, 1.0, 0.0) for r in range(_RPT)]

    bufs = (dbuf, dbuf2)
    sems = (semd, semd2)

    def chunk_cp(c):
        return pltpu.make_async_copy(
            x2d.at[pl.ds(dbase, _RPT), pl.ds(c * _BCW, _BCW)],
            bufs[c % 2], sems[c % 2])

    accs = tuple(jnp.zeros((_L,), jnp.float32) for _ in range(8))
    pending = chunk_cp(0)
    pending.start()
    for c in range(_NCH):
        if c + 1 < _NCH:
            nxt = chunk_cp(c + 1)
            nxt.start()
        pending.wait()
        cur = bufs[c % 2]

        def j_body(j, a, cur=cur):
            a = list(a)
            for r in range(_RPT):
                a[r % 8] = a[r % 8] + cur[r, pl.ds(j * _L, _L)] * mfs[r]
            return tuple(a)

        accs = lax.fori_loop(0, _BCW // _L, j_body, accs)
        if c + 1 < _NCH:
            pending = nxt
    total = accs[0]
    for q in range(1, 8):
        total = total + accs[q]

    pltpu.sync_copy(x2d.at[pl.ds(dbase, _RPT), pl.ds(0, 128)], c0buf)
    lane0 = jnp.where(lanes == 0, 1.0, 0.0)
    for r in range(_RPT):
        total = total - c0buf[r, pl.ds(0, _L)] * lane0 * mfs[r]
    tot_v[...] = total
    pltpu.sync_copy(tot_v, out_d.at[wid])


_sc_kernel = pl.kernel(
    _sc_body,
    out_type=[
        jax.ShapeDtypeStruct((_NW, _L), jnp.float32),
        jax.ShapeDtypeStruct((_NW, _L), jnp.float32),
        jax.ShapeDtypeStruct((_NW, _L), jnp.float32),
    ],
    mesh=plsc.VectorSubcoreMesh(core_axis_name="c", subcore_axis_name="s"),
    scratch_types=[
        pltpu.VMEM((_BPW,), jnp.int32),
        pltpu.VMEM((_BPW * _L,), jnp.float32),
        pltpu.VMEM((_L,), jnp.int32),
        pltpu.VMEM((_RPT, _BCW), jnp.float32),
        pltpu.VMEM((_RPT, _BCW), jnp.float32),
        pltpu.VMEM((_RPT, 128), jnp.float32),
        pltpu.VMEM((_L,), jnp.float32),
        pltpu.VMEM((_L,), jnp.float32),
        pltpu.VMEM((_L,), jnp.float32),
        pltpu.SemaphoreType.DMA,
        pltpu.SemaphoreType.DMA,
        pltpu.SemaphoreType.DMA,
    ],
)


def kernel(x, target):
    tgt = target.astype(jnp.int32)
    out_a, out_m, out_d = _sc_kernel(x, tgt)
    dense = pl.pallas_call(
        _dense_body,
        grid=(_R, _C),
        in_specs=[
            pl.BlockSpec((_BR, _BC), lambda r, c: (r, c)),
            pl.BlockSpec((1, 1, _BR), lambda r, c: (r, 0, 0)),
        ],
        out_specs=pl.BlockSpec((1, 1), lambda r, c: (0, 0)),
        out_shape=jax.ShapeDtypeStruct((1, 1), jnp.float32),
    )(x, tgt[:_TC_ROWS].reshape(_R, 1, _BR))
    a = jnp.sum(out_a)
    m = jnp.sum(out_m)
    d_sc = jnp.sum(out_d)
    return (_C_ROW * m - (_CONF - _EPS) * a - _EPS * d_sc
            + dense[0, 0]).reshape(())


# speedup vs baseline: 1.1142x; 1.0436x over previous
import math

import jax
import jax.numpy as jnp
from jax import lax
from jax.experimental import pallas as pl
from jax.experimental.pallas import tpu as pltpu
from jax.experimental.pallas import tpu_sc as plsc

_SIZE = 32000
_N = 2048
_EPS = 0.1 / (_SIZE - 2)
_CONF = 0.9
_C_ROW = _CONF * math.log(_CONF) + 0.1 * math.log(_EPS)

_BR = 256
_BC = 16000
_R = _N // _BR
_C = _SIZE // _BC

_INFO = plsc.get_sparse_core_info()
_NC, _NS, _L = _INFO.num_cores, _INFO.num_subcores, _INFO.num_lanes
_NW = _NC * _NS
_BPW = _N // _NW


def _dense_body(x_ref, tgt_ref, o_ref):
    r = pl.program_id(0)
    c = pl.program_id(1)
    first = jnp.logical_and(r == 0, c == 0)

    xb = x_ref[...]
    tgt = tgt_ref[0, 0, :]
    maskf = (tgt != 0).astype(jnp.float32)

    msum = jnp.sum(jnp.sum(xb, axis=1) * maskf)
    m0 = jnp.sum(xb[:, 0] * maskf)
    c0_term = jnp.where(c == 0, _EPS * m0, 0.0)

    contrib = (-_EPS * msum + c0_term).reshape(1, 1)
    o_ref[...] = jnp.where(first, contrib, o_ref[...] + contrib)


def _gather_body(x2d, tgt, out_a, out_m, tgt_v, win_v, acc_v, cnt_v, sem):
    wid = lax.axis_index("s") * _NC + lax.axis_index("c")
    base = wid * _BPW
    pltpu.sync_copy(tgt.at[pl.ds(base, _BPW)], tgt_v)
    lanes = lax.broadcasted_iota(jnp.int32, (_L,), 0)
    copies = []
    for k in range(_BPW // _L):
        t16 = tgt_v[pl.ds(k * _L, _L)]
        for l in range(_L):
            r = k * _L + l
            t_r = t16[l]
            cb = (t_r // _L) * _L
            copies.append(pltpu.make_async_copy(
                x2d.at[base + r, pl.ds(cb, _L)],
                win_v.at[pl.ds(r * _L, _L)],
                sem,
            ))
    for cp in copies:
        cp.start()
    for cp in copies:
        cp.wait()
    acc = jnp.zeros((_L,), jnp.float32)
    cnt = jnp.zeros((_L,), jnp.float32)
    for k in range(_BPW // _L):
        t16 = tgt_v[pl.ds(k * _L, _L)]
        m = t16 != 0
        cnt = cnt + jnp.where(m, 1.0, 0.0)
        for l in range(_L):
            r = k * _L + l
            t_r = t16[l]
            v = win_v[pl.ds(r * _L, _L)]
            sel_f = jnp.where(lanes == t_r % _L, 1.0, 0.0)
            mask_f = jnp.where(t_r != 0, 1.0, 0.0)
            acc = acc + v * sel_f * mask_f
    acc_v[...] = acc
    cnt_v[...] = cnt
    pltpu.sync_copy(acc_v, out_a.at[wid])
    pltpu.sync_copy(cnt_v, out_m.at[wid])


_sc_gather = pl.kernel(
    _gather_body,
    out_type=[
        jax.ShapeDtypeStruct((_NW, _L), jnp.float32),
        jax.ShapeDtypeStruct((_NW, _L), jnp.float32),
    ],
    mesh=plsc.VectorSubcoreMesh(core_axis_name="c", subcore_axis_name="s"),
    scratch_types=[
        pltpu.VMEM((_BPW,), jnp.int32),
        pltpu.VMEM((_BPW * _L,), jnp.float32),
        pltpu.VMEM((_L,), jnp.float32),
        pltpu.VMEM((_L,), jnp.float32),
        pltpu.SemaphoreType.DMA,
    ],
)


def kernel(x, target):
    tgt = target.astype(jnp.int32)
    out_a, out_m = _sc_gather(x, tgt)
    dense = pl.pallas_call(
        _dense_body,
        grid=(_R, _C),
        in_specs=[
            pl.BlockSpec((_BR, _BC), lambda r, c: (r, c)),
            pl.BlockSpec((1, 1, _BR), lambda r, c: (r, 0, 0)),
        ],
        out_specs=pl.BlockSpec((1, 1), lambda r, c: (0, 0)),
        out_shape=jax.ShapeDtypeStruct((1, 1), jnp.float32),
    )(x, tgt.reshape(_R, 1, _BR))
    a = jnp.sum(out_a)
    m = jnp.sum(out_m)
    return (_C_ROW * m - (_CONF - _EPS) * a + dense[0, 0]).reshape(())


# speedup vs baseline: 1.1163x; 1.0018x over previous
import math

import jax
import jax.numpy as jnp
from jax import lax
from jax.experimental import pallas as pl
from jax.experimental.pallas import tpu as pltpu
from jax.experimental.pallas import tpu_sc as plsc

_SIZE = 32000
_N = 2048
_EPS = 0.1 / (_SIZE - 2)
_CONF = 0.9
_C_ROW = _CONF * math.log(_CONF) + 0.1 * math.log(_EPS)

_BR = 128
_BC = 32000
_R = _N // _BR
_C = _SIZE // _BC

_INFO = plsc.get_sparse_core_info()
_NC, _NS, _L = _INFO.num_cores, _INFO.num_subcores, _INFO.num_lanes
_NW = _NC * _NS
_BPW = _N // _NW


def _dense_body(x_ref, tgt_ref, o_ref):
    r = pl.program_id(0)
    c = pl.program_id(1)
    first = jnp.logical_and(r == 0, c == 0)

    xb = x_ref[...]
    tgt = tgt_ref[0, 0, :]
    maskf = (tgt != 0).astype(jnp.float32)

    msum = jnp.sum(jnp.sum(xb, axis=1) * maskf)
    m0 = jnp.sum(xb[:, 0] * maskf)
    c0_term = jnp.where(c == 0, _EPS * m0, 0.0)

    contrib = (-_EPS * msum + c0_term).reshape(1, 1)
    o_ref[...] = jnp.where(first, contrib, o_ref[...] + contrib)


def _gather_body(x2d, tgt, out_a, out_m, tgt_v, win_v, acc_v, cnt_v, sem):
    wid = lax.axis_index("s") * _NC + lax.axis_index("c")
    base = wid * _BPW
    pltpu.sync_copy(tgt.at[pl.ds(base, _BPW)], tgt_v)
    lanes = lax.broadcasted_iota(jnp.int32, (_L,), 0)
    copies = []
    for k in range(_BPW // _L):
        t16 = tgt_v[pl.ds(k * _L, _L)]
        for l in range(_L):
            r = k * _L + l
            t_r = t16[l]
            cb = (t_r // _L) * _L
            copies.append(pltpu.make_async_copy(
                x2d.at[base + r, pl.ds(cb, _L)],
                win_v.at[pl.ds(r * _L, _L)],
                sem,
            ))
    for cp in copies:
        cp.start()
    for cp in copies:
        cp.wait()
    acc = jnp.zeros((_L,), jnp.float32)
    cnt = jnp.zeros((_L,), jnp.float32)
    for k in range(_BPW // _L):
        t16 = tgt_v[pl.ds(k * _L, _L)]
        m = t16 != 0
        cnt = cnt + jnp.where(m, 1.0, 0.0)
        for l in range(_L):
            r = k * _L + l
            t_r = t16[l]
            v = win_v[pl.ds(r * _L, _L)]
            sel_f = jnp.where(lanes == t_r % _L, 1.0, 0.0)
            mask_f = jnp.where(t_r != 0, 1.0, 0.0)
            acc = acc + v * sel_f * mask_f
    acc_v[...] = acc
    cnt_v[...] = cnt
    pltpu.sync_copy(acc_v, out_a.at[wid])
    pltpu.sync_copy(cnt_v, out_m.at[wid])


_sc_gather = pl.kernel(
    _gather_body,
    out_type=[
        jax.ShapeDtypeStruct((_NW, _L), jnp.float32),
        jax.ShapeDtypeStruct((_NW, _L), jnp.float32),
    ],
    mesh=plsc.VectorSubcoreMesh(core_axis_name="c", subcore_axis_name="s"),
    scratch_types=[
        pltpu.VMEM((_BPW,), jnp.int32),
        pltpu.VMEM((_BPW * _L,), jnp.float32),
        pltpu.VMEM((_L,), jnp.float32),
        pltpu.VMEM((_L,), jnp.float32),
        pltpu.SemaphoreType.DMA,
    ],
)


def kernel(x, target):
    tgt = target.astype(jnp.int32)
    out_a, out_m = _sc_gather(x, tgt)
    dense = pl.pallas_call(
        _dense_body,
        grid=(_R, _C),
        in_specs=[
            pl.BlockSpec((_BR, _BC), lambda r, c: (r, c)),
            pl.BlockSpec((1, 1, _BR), lambda r, c: (r, 0, 0)),
        ],
        out_specs=pl.BlockSpec((1, 1), lambda r, c: (0, 0)),
        out_shape=jax.ShapeDtypeStruct((1, 1), jnp.float32),
    )(x, tgt.reshape(_R, 1, _BR))
    a = jnp.sum(out_a)
    m = jnp.sum(out_m)
    return (_C_ROW * m - (_CONF - _EPS) * a + dense[0, 0]).reshape(())
